# Initial kernel scaffold; baseline (speedup 1.0000x reference)
import jax
import jax.numpy as jnp
from jax.experimental import pallas as pl


def kernel(tensor):
    raise NotImplementedError("write your pallas kernel here")


# speedup vs baseline: 2.2724x; 2.2724x over previous
import jax
import jax.numpy as jnp
from jax import lax
from jax.experimental import pallas as pl
from jax.experimental.pallas import tpu as pltpu
from jax.experimental.pallas import tpu_sc as plsc

LM = 128
W = 2 * LM - 1
PACKED = LM * LM
OUT_FLAT = LM * W
BATCH = 1024
NW = 32
ROWS = BATCH // NW

IN_PAD = PACKED + 16
OUT_PAD = OUT_FLAT + 16


def _body(in_hbm, out_hbm, in_v, out_v):
    wid = lax.axis_index("s") * 2 + lax.axis_index("c")

    def zero(i, c):
        out_v[pl.ds(16 * i, 16)] = jnp.zeros((16,), jnp.float32)
        return c

    lax.fori_loop(0, OUT_PAD // 16, zero, 0)

    lane = lax.iota(jnp.int32, 16)

    def row(r, c):
        b = wid * ROWS + r
        pltpu.sync_copy(in_hbm.at[b], in_v.at[pl.ds(0, PACKED)])

        def lbody(l, c2):
            n = 2 * l + 1
            src = l * l
            dst = 254 * l + 127
            nv = l // 8 + 1

            def kbody(k, c3):
                off = 16 * k
                v = in_v[pl.ds(src + off, 16)]
                m = (lane + off) < n
                out_v[pl.ds(dst + off, 16)] = jnp.where(m, v, 0.0)
                return c3

            lax.fori_loop(0, nv, kbody, 0)
            return c2

        lax.fori_loop(0, LM, lbody, 0)
        pltpu.sync_copy(out_v.at[pl.ds(0, OUT_FLAT)], out_hbm.at[b])
        return c

    lax.fori_loop(0, ROWS, row, 0)


def kernel(tensor):
    mesh = plsc.VectorSubcoreMesh(core_axis_name="c", subcore_axis_name="s")
    k = pl.kernel(
        _body,
        mesh=mesh,
        out_type=jax.ShapeDtypeStruct((BATCH, OUT_FLAT), jnp.float32),
        scratch_types=[
            pltpu.VMEM((IN_PAD,), jnp.float32),
            pltpu.VMEM((OUT_PAD,), jnp.float32),
        ],
    )
    out = k(tensor)
    return out.reshape(BATCH, LM, W)


# speedup vs baseline: 2.8732x; 1.2644x over previous
import jax
import jax.numpy as jnp
from jax import lax
from jax.experimental import pallas as pl
from jax.experimental.pallas import tpu as pltpu
from jax.experimental.pallas import tpu_sc as plsc

LM = 128
W = 2 * LM - 1
PACKED = LM * LM
OUT_FLAT = LM * W
BATCH = 1024
NW = 32
ROWS = BATCH // NW

IN_PAD = PACKED + 16
OUT_PAD = OUT_FLAT + 16


def _copy_row(in_v, out_v, lane):
    for g in range(16):
        def lbody(l, c, g=g):
            src = l * l
            dst = 254 * l + 127
            for k in range(g):
                out_v[pl.ds(dst + 16 * k, 16)] = in_v[pl.ds(src + 16 * k, 16)]
            rem = 2 * l + 1 - 16 * g
            v = in_v[pl.ds(src + 16 * g, 16)]
            out_v[pl.ds(dst + 16 * g, 16)] = jnp.where(lane < rem, v, 0.0)
            return c

        lax.fori_loop(8 * g, 8 * g + 8, lbody, 0)


def _body(in_hbm, out_hbm, in0, in1, out0, out1, si0, si1, so0, so1):
    wid = lax.axis_index("s") * 2 + lax.axis_index("c")
    base = wid * ROWS
    ins = (in0, in1)
    outs = (out0, out1)
    sis = (si0, si1)
    sos = (so0, so1)
    lane = lax.iota(jnp.int32, 16)

    def zero(i, c):
        out0[pl.ds(16 * i, 16)] = jnp.zeros((16,), jnp.float32)
        out1[pl.ds(16 * i, 16)] = jnp.zeros((16,), jnp.float32)
        return c

    lax.fori_loop(0, OUT_PAD // 16, zero, 0)

    for p in range(2):
        pltpu.make_async_copy(
            in_hbm.at[base + p], ins[p].at[pl.ds(0, PACKED)], sis[p]
        ).start()

    def pair(i, c):
        for p in range(2):
            b = base + 2 * i + p
            pltpu.make_async_copy(
                in_hbm.at[b], ins[p].at[pl.ds(0, PACKED)], sis[p]
            ).wait()

            @pl.when(i > 0)
            def _():
                pltpu.make_async_copy(
                    outs[p].at[pl.ds(0, OUT_FLAT)], out_hbm.at[b - 2], sos[p]
                ).wait()

            _copy_row(ins[p], outs[p], lane)
            pltpu.make_async_copy(
                outs[p].at[pl.ds(0, OUT_FLAT)], out_hbm.at[b], sos[p]
            ).start()

            @pl.when(2 * i + p + 2 < ROWS)
            def _():
                pltpu.make_async_copy(
                    in_hbm.at[b + 2], ins[p].at[pl.ds(0, PACKED)], sis[p]
                ).start()
        return c

    lax.fori_loop(0, ROWS // 2, pair, 0)

    for p in range(2):
        pltpu.make_async_copy(
            outs[p].at[pl.ds(0, OUT_FLAT)], out_hbm.at[base + ROWS - 2 + p], sos[p]
        ).wait()


def kernel(tensor):
    mesh = plsc.VectorSubcoreMesh(core_axis_name="c", subcore_axis_name="s")
    k = pl.kernel(
        _body,
        mesh=mesh,
        out_type=jax.ShapeDtypeStruct((BATCH, OUT_FLAT), jnp.float32),
        scratch_types=[
            pltpu.VMEM((IN_PAD,), jnp.float32),
            pltpu.VMEM((IN_PAD,), jnp.float32),
            pltpu.VMEM((OUT_PAD,), jnp.float32),
            pltpu.VMEM((OUT_PAD,), jnp.float32),
            pltpu.SemaphoreType.DMA,
            pltpu.SemaphoreType.DMA,
            pltpu.SemaphoreType.DMA,
            pltpu.SemaphoreType.DMA,
        ],
    )
    out = k(tensor)
    return out.reshape(BATCH, LM, W)


# speedup vs baseline: 3.3239x; 1.1569x over previous
import jax
import jax.numpy as jnp
from jax import lax
from jax.experimental import pallas as pl
from jax.experimental.pallas import tpu as pltpu
from jax.experimental.pallas import tpu_sc as plsc

LM = 128
W = 2 * LM - 1
PACKED = LM * LM
BATCH = 1024
NW = 32
ROWS = BATCH // NW

IN_OFF = 16
IN_PAD = PACKED + 32
OUT_ROWS = LM + 1


def _copy_row(in_v, out_v, lane):
    def lbody(l, c):
        src0 = l * l + l - 127 + IN_OFF
        k0 = (127 - l) // 16
        k1 = (127 + l) // 16

        def masked(k):
            col = pl.multiple_of(16 * k, 16)
            v = in_v[pl.ds(src0 + col, 16)]
            m = jnp.abs(lane + (col - 127)) <= l
            out_v[l, pl.ds(col, 16)] = jnp.where(m, v, 0.0)

        masked(k0)

        @pl.when(k1 > k0)
        def _():
            masked(k1)

        def kin(k, c2):
            col = pl.multiple_of(16 * k, 16)
            out_v[l, pl.ds(col, 16)] = in_v[pl.ds(src0 + col, 16)]
            return c2

        lax.fori_loop(k0 + 1, k1, kin, 0)
        return c

    lax.fori_loop(0, LM, lbody, 0)


def _body(in_hbm, out_hbm, in0, in1, out0, out1, si0, si1, so0, so1):
    wid = lax.axis_index("s") * 2 + lax.axis_index("c")
    base = wid * ROWS
    ins = (in0, in1)
    outs = (out0, out1)
    sis = (si0, si1)
    sos = (so0, so1)
    lane = lax.iota(jnp.int32, 16)
    zeros = jnp.zeros((16,), jnp.float32)

    def zero(l, c):
        def zcol(k, c2):
            col = pl.multiple_of(16 * k, 16)
            out0[l, pl.ds(col, 16)] = zeros
            out1[l, pl.ds(col, 16)] = zeros
            return c2

        lax.fori_loop(0, 16, zcol, 0)
        return c

    lax.fori_loop(0, LM, zero, 0)

    for p in range(2):
        pltpu.make_async_copy(
            in_hbm.at[pl.ds((base + p) * PACKED, PACKED)], ins[p].at[pl.ds(IN_OFF, PACKED)], sis[p]
        ).start()

    def pair(i, c):
        for p in range(2):
            b = base + 2 * i + p
            pltpu.make_async_copy(
                in_hbm.at[pl.ds(b * PACKED, PACKED)], ins[p].at[pl.ds(IN_OFF, PACKED)], sis[p]
            ).wait()

            @pl.when(i > 0)
            def _():
                pltpu.make_async_copy(
                    outs[p].at[pl.ds(0, LM)], out_hbm.at[b - 2], sos[p]
                ).wait()

            _copy_row(ins[p], outs[p], lane)
            pltpu.make_async_copy(
                outs[p].at[pl.ds(0, LM)], out_hbm.at[b], sos[p]
            ).start()

            @pl.when(2 * i + p + 2 < ROWS)
            def _():
                pltpu.make_async_copy(
                    in_hbm.at[pl.ds((b + 2) * PACKED, PACKED)], ins[p].at[pl.ds(IN_OFF, PACKED)], sis[p]
                ).start()
        return c

    lax.fori_loop(0, ROWS // 2, pair, 0)

    for p in range(2):
        pltpu.make_async_copy(
            outs[p].at[pl.ds(0, LM)], out_hbm.at[base + ROWS - 2 + p], sos[p]
        ).wait()


def kernel(tensor):
    mesh = plsc.VectorSubcoreMesh(core_axis_name="c", subcore_axis_name="s")
    k = pl.kernel(
        _body,
        mesh=mesh,
        out_type=jax.ShapeDtypeStruct((BATCH, LM, W), jnp.float32),
        scratch_types=[
            pltpu.VMEM((IN_PAD,), jnp.float32),
            pltpu.VMEM((IN_PAD,), jnp.float32),
            pltpu.VMEM((OUT_ROWS, W), jnp.float32),
            pltpu.VMEM((OUT_ROWS, W), jnp.float32),
            pltpu.SemaphoreType.DMA,
            pltpu.SemaphoreType.DMA,
            pltpu.SemaphoreType.DMA,
            pltpu.SemaphoreType.DMA,
        ],
    )
    return k(tensor.reshape(BATCH * PACKED))


# speedup vs baseline: 3.8215x; 1.1497x over previous
import jax
import jax.numpy as jnp
from jax import lax
from jax.experimental import pallas as pl
from jax.experimental.pallas import tpu as pltpu
from jax.experimental.pallas import tpu_sc as plsc

LM = 128
W = 2 * LM - 1
PACKED = LM * LM
BATCH = 1024
NW = 32
ROWS = BATCH // NW

IN_OFF = 16
IN_PAD = PACKED + 32
OUT_ROWS = LM + 1


def _copy_pair(inA, inB, outA, outB, lane):
    def lbody(l, c):
        src0 = l * l + l - 127 + IN_OFF
        k0 = (127 - l) >> 4
        k1 = (127 + l) >> 4

        def masked(out_v, in_v, k):
            col = pl.multiple_of(16 * k, 16)
            v = in_v[pl.ds(src0 + col, 16)]
            m = jnp.abs(lane + (col - 127)) <= l
            out_v[l, pl.ds(col, 16)] = jnp.where(m, v, 0.0)

        masked(outA, inA, k0)
        masked(outB, inB, k0)
        masked(outA, inA, k1)
        masked(outB, inB, k1)

        @plsc.parallel_loop(k0 + 1, k1, unroll=4)
        def _(k):
            col = pl.multiple_of(16 * k, 16)
            outA[l, pl.ds(col, 16)] = inA[pl.ds(src0 + col, 16)]
            outB[l, pl.ds(col, 16)] = inB[pl.ds(src0 + col, 16)]

        return c

    lax.fori_loop(0, LM, lbody, 0)


def _body(in_hbm, out_hbm, in0, in1, out0, out1, si0, si1, so0, so1):
    wid = lax.axis_index("s") * 2 + lax.axis_index("c")
    base = wid * ROWS
    ins = (in0, in1)
    outs = (out0, out1)
    sis = (si0, si1)
    sos = (so0, so1)
    lane = lax.iota(jnp.int32, 16)
    zeros = jnp.zeros((16,), jnp.float32)

    def zero(l, c):
        def zcol(k, c2):
            col = pl.multiple_of(16 * k, 16)
            out0[l, pl.ds(col, 16)] = zeros
            out1[l, pl.ds(col, 16)] = zeros
            return c2

        lax.fori_loop(0, 16, zcol, 0)
        return c

    lax.fori_loop(0, LM, zero, 0)

    for p in range(2):
        pltpu.make_async_copy(
            in_hbm.at[pl.ds((base + p) * PACKED, PACKED)], ins[p].at[pl.ds(IN_OFF, PACKED)], sis[p]
        ).start()

    def pair(i, c):
        b = base + 2 * i
        for p in range(2):
            pltpu.make_async_copy(
                in_hbm.at[pl.ds((b + p) * PACKED, PACKED)],
                ins[p].at[pl.ds(IN_OFF, PACKED)], sis[p]
            ).wait()

        @pl.when(i > 0)
        def _():
            for p in range(2):
                pltpu.make_async_copy(
                    outs[p].at[pl.ds(0, LM)], out_hbm.at[b - 2 + p], sos[p]
                ).wait()

        _copy_pair(in0, in1, out0, out1, lane)

        for p in range(2):
            pltpu.make_async_copy(
                outs[p].at[pl.ds(0, LM)], out_hbm.at[b + p], sos[p]
            ).start()

        @pl.when(2 * i + 2 < ROWS)
        def _():
            for p in range(2):
                pltpu.make_async_copy(
                    in_hbm.at[pl.ds((b + 2 + p) * PACKED, PACKED)],
                    ins[p].at[pl.ds(IN_OFF, PACKED)], sis[p]
                ).start()
        return c

    lax.fori_loop(0, ROWS // 2, pair, 0)

    for p in range(2):
        pltpu.make_async_copy(
            outs[p].at[pl.ds(0, LM)], out_hbm.at[base + ROWS - 2 + p], sos[p]
        ).wait()


def kernel(tensor):
    mesh = plsc.VectorSubcoreMesh(core_axis_name="c", subcore_axis_name="s")
    k = pl.kernel(
        _body,
        mesh=mesh,
        out_type=jax.ShapeDtypeStruct((BATCH, LM, W), jnp.float32),
        scratch_types=[
            pltpu.VMEM((IN_PAD,), jnp.float32),
            pltpu.VMEM((IN_PAD,), jnp.float32),
            pltpu.VMEM((OUT_ROWS, W), jnp.float32),
            pltpu.VMEM((OUT_ROWS, W), jnp.float32),
            pltpu.SemaphoreType.DMA,
            pltpu.SemaphoreType.DMA,
            pltpu.SemaphoreType.DMA,
            pltpu.SemaphoreType.DMA,
        ],
    )
    return k(tensor.reshape(BATCH * PACKED))
